# baseline (speedup 1.0000x reference)
import functools

import jax
import jax.numpy as jnp
import numpy as np
from jax.experimental import pallas as pl
from jax.experimental.pallas import tpu as pltpu

_EPS = 1e-6
_BN_EPS = 1e-5
_NEG = 0.2


def _vn_layer(pd, c, istd, mistd):
    p = [pd[v * c:(v + 1) * c] for v in range(3)]
    d = [pd[(3 + v) * c:(4 + v) * c] for v in range(3)]
    nsq = p[0] * p[0] + p[1] * p[1] + p[2] * p[2]
    scale = istd - mistd * jax.lax.rsqrt(nsq + _EPS * _EPS)
    p = [pv * scale for pv in p]
    dotp = p[0] * d[0] + p[1] * d[1] + p[2] * d[2]
    dsq = d[0] * d[0] + d[1] * d[1] + d[2] * d[2]
    fac = (1.0 - _NEG) * (jnp.minimum(dotp, 0.0) / (dsq + _EPS))
    return [p[v] - fac * d[v] for v in range(3)]


def _norm_moments(p, c):
    norm = jnp.sqrt(p[0:c] * p[0:c] + p[c:2 * c] * p[c:2 * c]
                    + p[2 * c:] * p[2 * c:]) + _EPS
    s = jnp.sum(norm, axis=1, keepdims=True)
    ss = jnp.sum(norm * norm, axis=1, keepdims=True)
    return jnp.concatenate([s, ss], axis=0)


def _bf16_cat(parts):
    return jnp.concatenate([t.astype(jnp.bfloat16) for t in parts], axis=0)


def _pass1_kernel(x_ref, w1f_ref, xbf_ref, mom_ref, *, c1):
    acc = 0.0
    for i in range(x_ref.shape[0]):
        xb = x_ref[i].astype(jnp.bfloat16)
        xbf_ref[i] = xb
        p = jnp.dot(w1f_ref[...], xb, preferred_element_type=jnp.float32)
        acc = acc + _norm_moments(p, c1)
    mom_ref[0] = acc


def _pass2_kernel(xbf_ref, w1_ref, s1_ref, ms1_ref, w2f_ref, mom_ref, *,
                  c1, c2):
    acc = 0.0
    for i in range(xbf_ref.shape[0]):
        pd1 = jnp.dot(w1_ref[...], xbf_ref[i],
                      preferred_element_type=jnp.float32)
        q1 = _bf16_cat(_vn_layer(pd1, c1, s1_ref[...], ms1_ref[...]))
        p2 = jnp.dot(w2f_ref[...], q1, preferred_element_type=jnp.float32)
        acc = acc + _norm_moments(p2, c2)
    mom_ref[0] = acc


def _pass3_kernel(xbf_ref, w1_ref, s1_ref, ms1_ref, w2_ref, s2_ref, ms2_ref,
                  wl_ref, xstd_ref, z_ref, *, c, c1, c2):
    for i in range(xbf_ref.shape[0]):
        xb = xbf_ref[i]
        pd1 = jnp.dot(w1_ref[...], xb, preferred_element_type=jnp.float32)
        q1 = _bf16_cat(_vn_layer(pd1, c1, s1_ref[...], ms1_ref[...]))
        pd2 = jnp.dot(w2_ref[...], q1, preferred_element_type=jnp.float32)
        q2 = _bf16_cat(_vn_layer(pd2, c2, s2_ref[...], ms2_ref[...]))
        z = jnp.dot(wl_ref[...], q2,
                    preferred_element_type=jnp.float32)
        z_ref[i] = z
        xv = [xb[v * c:(v + 1) * c].astype(jnp.float32) for v in range(3)]
        for k in range(3):
            xstd_ref[i, k * c:(k + 1) * c] = (
                xv[0] * z[k:k + 1] + xv[1] * z[3 + k:4 + k]
                + xv[2] * z[6 + k:7 + k])


def _build_weights(wf1, wd1, wf2, wd2, wlin, c, c1, c2):
    i3 = np.eye(3, dtype=np.float32)
    bf = jnp.bfloat16

    def blk(w):
        co, ci = w.shape
        return jnp.einsum('vu,cd->vcud', i3, w).reshape(3 * co, 3 * ci)

    w1f = blk(wf1).astype(bf)
    w1 = jnp.concatenate([blk(wf1), blk(wd1)], axis=0).astype(bf)
    w2f = blk(wf2).astype(bf)
    w2 = jnp.concatenate([blk(wf2), blk(wd2)], axis=0).astype(bf)
    wl = jnp.einsum('vu,kc->vkuc', i3, wlin).reshape(9, 3 * c2).astype(bf)
    return w1f, w1, w2f, w2, wl


def kernel(x, wf1, wd1, wf2, wd2, wlin):
    b, c, v, n = x.shape
    assert v == 3
    c1, c2 = wf1.shape[0], wf2.shape[0]
    f32 = jnp.float32
    xm = jnp.transpose(x.astype(f32), (0, 2, 1, 3)).reshape(b, 3 * c, n)

    w1f, w1, w2f, w2, wl = _build_weights(wf1, wd1, wf2, wd2, wlin, c, c1, c2)

    total = float(b * n)

    bb = 8 if b % 8 == 0 else 1
    g = b // bb
    xspec = pl.BlockSpec((bb, 3 * c, n), lambda i: (i, 0, 0))

    def wspec(a):
        zeros = (0,) * a.ndim
        return pl.BlockSpec(a.shape, lambda i: zeros)

    def momspec(ch):
        return pl.BlockSpec((1, 2 * ch, 1), lambda i: (i, 0, 0))

    params = pltpu.CompilerParams(dimension_semantics=("parallel",),
                                  vmem_limit_bytes=50 * 1024 * 1024)

    def finalize(mom, ch):
        s = jnp.sum(mom[:, :ch, 0], axis=0)
        ss = jnp.sum(mom[:, ch:, 0], axis=0)
        mean = s / total
        var = ss / total - mean * mean
        istd = jax.lax.rsqrt(var + _BN_EPS)
        return istd.reshape(ch, 1), (mean * istd).reshape(ch, 1)

    xbf, mom1 = pl.pallas_call(
        functools.partial(_pass1_kernel, c1=c1),
        grid=(g,),
        in_specs=[xspec, wspec(w1f)],
        out_specs=(xspec, momspec(c1)),
        out_shape=(jax.ShapeDtypeStruct((b, 3 * c, n), jnp.bfloat16),
                   jax.ShapeDtypeStruct((g, 2 * c1, 1), f32)),
        compiler_params=params,
    )(xm, w1f)
    istd1, mistd1 = finalize(mom1, c1)

    mom2 = pl.pallas_call(
        functools.partial(_pass2_kernel, c1=c1, c2=c2),
        grid=(g,),
        in_specs=[xspec, wspec(w1), wspec(istd1), wspec(mistd1), wspec(w2f)],
        out_specs=momspec(c2),
        out_shape=jax.ShapeDtypeStruct((g, 2 * c2, 1), f32),
        compiler_params=params,
    )(xbf, w1, istd1, mistd1, w2f)
    istd2, mistd2 = finalize(mom2, c2)

    xstd_m, z_m = pl.pallas_call(
        functools.partial(_pass3_kernel, c=c, c1=c1, c2=c2),
        grid=(g,),
        in_specs=[xspec, wspec(w1), wspec(istd1), wspec(mistd1), wspec(w2),
                  wspec(istd2), wspec(mistd2), wspec(wl)],
        out_specs=(xspec, pl.BlockSpec((bb, 9, n), lambda i: (i, 0, 0))),
        out_shape=(jax.ShapeDtypeStruct((b, 3 * c, n), f32),
                   jax.ShapeDtypeStruct((b, 9, n), f32)),
        compiler_params=params,
    )(xbf, w1, istd1, mistd1, w2, istd2, mistd2, wl)

    x_std = jnp.transpose(xstd_m.reshape(b, 3, c, n), (0, 2, 1, 3))
    z0 = z_m.reshape(b, 3, 3, n)
    return x_std, z0


# speedup vs baseline: 1.0179x; 1.0179x over previous
import functools

import jax
import jax.numpy as jnp
import numpy as np
from jax.experimental import pallas as pl
from jax.experimental.pallas import tpu as pltpu

_EPS = 1e-6
_BN_EPS = 1e-5
_NEG = 0.2


def _vn_layer(pd, c, istd, mistd):
    p = [pd[v * c:(v + 1) * c] for v in range(3)]
    d = [pd[(3 + v) * c:(4 + v) * c] for v in range(3)]
    nsq = p[0] * p[0] + p[1] * p[1] + p[2] * p[2]
    scale = istd - mistd * jax.lax.rsqrt(nsq + _EPS * _EPS)
    p = [pv * scale for pv in p]
    dotp = p[0] * d[0] + p[1] * d[1] + p[2] * d[2]
    dsq = d[0] * d[0] + d[1] * d[1] + d[2] * d[2]
    fac = (1.0 - _NEG) * (jnp.minimum(dotp, 0.0) / (dsq + _EPS))
    return [p[v] - fac * d[v] for v in range(3)]


def _norm_moments(p, c):
    norm = jnp.sqrt(p[0:c] * p[0:c] + p[c:2 * c] * p[c:2 * c]
                    + p[2 * c:] * p[2 * c:]) + _EPS
    s = jnp.sum(norm, axis=1, keepdims=True)
    ss = jnp.sum(norm * norm, axis=1, keepdims=True)
    return jnp.concatenate([s, ss], axis=0)


def _bf16_cat(parts):
    return jnp.concatenate([t.astype(jnp.bfloat16) for t in parts], axis=0)


def _pass1_kernel(x_ref, w1f_ref, xbf_ref, mom_ref, *, c1):
    acc = 0.0
    for i in range(x_ref.shape[0]):
        xb = x_ref[i].astype(jnp.bfloat16)
        xbf_ref[i] = xb
        p = jnp.dot(w1f_ref[...], xb, preferred_element_type=jnp.float32)
        acc = acc + _norm_moments(p, c1)
    mom_ref[0] = acc


def _pass2_kernel(xbf_ref, w1_ref, s1_ref, ms1_ref, w2f_ref, mom_ref, *,
                  c1, c2):
    acc = 0.0
    for i in range(xbf_ref.shape[0]):
        pd1 = jnp.dot(w1_ref[...], xbf_ref[i],
                      preferred_element_type=jnp.float32)
        q1 = _bf16_cat(_vn_layer(pd1, c1, s1_ref[...], ms1_ref[...]))
        p2 = jnp.dot(w2f_ref[...], q1, preferred_element_type=jnp.float32)
        acc = acc + _norm_moments(p2, c2)
    mom_ref[0] = acc


def _pass3_kernel(xbf_ref, w1_ref, s1_ref, ms1_ref, w2_ref, s2_ref, ms2_ref,
                  wl_ref, xstd_ref, z_ref, *, c, c1, c2):
    for i in range(xbf_ref.shape[0]):
        xb = xbf_ref[i]
        pd1 = jnp.dot(w1_ref[...], xb, preferred_element_type=jnp.float32)
        q1 = _bf16_cat(_vn_layer(pd1, c1, s1_ref[...], ms1_ref[...]))
        pd2 = jnp.dot(w2_ref[...], q1, preferred_element_type=jnp.float32)
        q2 = _bf16_cat(_vn_layer(pd2, c2, s2_ref[...], ms2_ref[...]))
        z = jnp.dot(wl_ref[...], q2,
                    preferred_element_type=jnp.float32)
        z_ref[i] = z
        xv = [xb[v * c:(v + 1) * c].astype(jnp.float32) for v in range(3)]
        for k in range(3):
            xstd_ref[i, k * c:(k + 1) * c] = (
                xv[0] * z[k:k + 1] + xv[1] * z[3 + k:4 + k]
                + xv[2] * z[6 + k:7 + k])


def _build_weights(wf1, wd1, wf2, wd2, wlin, c, c1, c2):
    i3 = np.eye(3, dtype=np.float32)
    bf = jnp.bfloat16

    def blk(w):
        co, ci = w.shape
        return jnp.einsum('vu,cd->vcud', i3, w).reshape(3 * co, 3 * ci)

    w1f = blk(wf1).astype(bf)
    w1 = jnp.concatenate([blk(wf1), blk(wd1)], axis=0).astype(bf)
    w2f = blk(wf2).astype(bf)
    w2 = jnp.concatenate([blk(wf2), blk(wd2)], axis=0).astype(bf)
    wl = jnp.einsum('vu,kc->vkuc', i3, wlin).reshape(9, 3 * c2).astype(bf)
    return w1f, w1, w2f, w2, wl


def kernel(x, wf1, wd1, wf2, wd2, wlin):
    b, c, v, n = x.shape
    assert v == 3
    c1, c2 = wf1.shape[0], wf2.shape[0]
    f32 = jnp.float32
    xm = jnp.transpose(x.astype(f32), (0, 2, 1, 3)).reshape(b, 3 * c, n)

    w1f, w1, w2f, w2, wl = _build_weights(wf1, wd1, wf2, wd2, wlin, c, c1, c2)

    total = float(b * n)

    bb = 4 if b % 4 == 0 else 1
    g = b // bb
    xspec = pl.BlockSpec((bb, 3 * c, n), lambda i: (i, 0, 0))

    def wspec(a):
        zeros = (0,) * a.ndim
        return pl.BlockSpec(a.shape, lambda i: zeros)

    def momspec(ch):
        return pl.BlockSpec((1, 2 * ch, 1), lambda i: (i, 0, 0))

    params = pltpu.CompilerParams(dimension_semantics=("parallel",),
                                  vmem_limit_bytes=50 * 1024 * 1024)

    def finalize(mom, ch):
        s = jnp.sum(mom[:, :ch, 0], axis=0)
        ss = jnp.sum(mom[:, ch:, 0], axis=0)
        mean = s / total
        var = ss / total - mean * mean
        istd = jax.lax.rsqrt(var + _BN_EPS)
        return istd.reshape(ch, 1), (mean * istd).reshape(ch, 1)

    xbf, mom1 = pl.pallas_call(
        functools.partial(_pass1_kernel, c1=c1),
        grid=(g,),
        in_specs=[xspec, wspec(w1f)],
        out_specs=(xspec, momspec(c1)),
        out_shape=(jax.ShapeDtypeStruct((b, 3 * c, n), jnp.bfloat16),
                   jax.ShapeDtypeStruct((g, 2 * c1, 1), f32)),
        compiler_params=params,
    )(xm, w1f)
    istd1, mistd1 = finalize(mom1, c1)

    mom2 = pl.pallas_call(
        functools.partial(_pass2_kernel, c1=c1, c2=c2),
        grid=(g,),
        in_specs=[xspec, wspec(w1), wspec(istd1), wspec(mistd1), wspec(w2f)],
        out_specs=momspec(c2),
        out_shape=jax.ShapeDtypeStruct((g, 2 * c2, 1), f32),
        compiler_params=params,
    )(xbf, w1, istd1, mistd1, w2f)
    istd2, mistd2 = finalize(mom2, c2)

    xstd_m, z_m = pl.pallas_call(
        functools.partial(_pass3_kernel, c=c, c1=c1, c2=c2),
        grid=(g,),
        in_specs=[xspec, wspec(w1), wspec(istd1), wspec(mistd1), wspec(w2),
                  wspec(istd2), wspec(mistd2), wspec(wl)],
        out_specs=(xspec, pl.BlockSpec((bb, 9, n), lambda i: (i, 0, 0))),
        out_shape=(jax.ShapeDtypeStruct((b, 3 * c, n), f32),
                   jax.ShapeDtypeStruct((b, 9, n), f32)),
        compiler_params=params,
    )(xbf, w1, istd1, mistd1, w2, istd2, mistd2, wl)

    x_std = jnp.transpose(xstd_m.reshape(b, 3, c, n), (0, 2, 1, 3))
    z0 = z_m.reshape(b, 3, 3, n)
    return x_std, z0


# speedup vs baseline: 1.0976x; 1.0783x over previous
import functools

import jax
import jax.numpy as jnp
import numpy as np
from jax.experimental import pallas as pl
from jax.experimental.pallas import tpu as pltpu

_EPS = 1e-6
_BN_EPS = 1e-5
_NEG = 0.2


def _vn_layer(pd, c, istd, mistd):
    p = [pd[v * c:(v + 1) * c] for v in range(3)]
    d = [pd[(3 + v) * c:(4 + v) * c] for v in range(3)]
    nsq = p[0] * p[0] + p[1] * p[1] + p[2] * p[2]
    scale = istd - mistd * jax.lax.rsqrt(nsq + _EPS * _EPS)
    p = [pv * scale for pv in p]
    dotp = p[0] * d[0] + p[1] * d[1] + p[2] * d[2]
    dsq = d[0] * d[0] + d[1] * d[1] + d[2] * d[2]
    fac = (1.0 - _NEG) * (jnp.minimum(dotp, 0.0) / (dsq + _EPS))
    return [p[v] - fac * d[v] for v in range(3)]


def _norm_moments(p, c):
    norm = jnp.sqrt(p[0:c] * p[0:c] + p[c:2 * c] * p[c:2 * c]
                    + p[2 * c:] * p[2 * c:]) + _EPS
    s = jnp.sum(norm, axis=1, keepdims=True)
    ss = jnp.sum(norm * norm, axis=1, keepdims=True)
    return jnp.concatenate([s, ss], axis=0)


def _bf16_cat(parts):
    return jnp.concatenate([t.astype(jnp.bfloat16) for t in parts], axis=0)


def _pass1_kernel(x_ref, w1f_ref, xbf_ref, mom_ref, *, c1):
    acc = 0.0
    for i in range(x_ref.shape[0]):
        xb = x_ref[i].astype(jnp.bfloat16)
        xbf_ref[i] = xb
        p = jnp.dot(w1f_ref[...], xb, preferred_element_type=jnp.float32)
        acc = acc + _norm_moments(p, c1)
    mom_ref[0] = acc


def _pass2_kernel(xbf_ref, w1_ref, s1_ref, ms1_ref, w2f_ref, mom_ref, *,
                  c1, c2):
    acc = 0.0
    for i in range(xbf_ref.shape[0]):
        pd1 = jnp.dot(w1_ref[...], xbf_ref[i],
                      preferred_element_type=jnp.float32)
        q1 = _bf16_cat(_vn_layer(pd1, c1, s1_ref[...], ms1_ref[...]))
        p2 = jnp.dot(w2f_ref[...], q1, preferred_element_type=jnp.float32)
        acc = acc + _norm_moments(p2, c2)
    mom_ref[0] = acc


def _pass3_kernel(xbf_ref, w1_ref, s1_ref, ms1_ref, w2_ref, s2_ref, ms2_ref,
                  wl_ref, xstd_ref, z_ref, *, c, c1, c2):
    for i in range(xbf_ref.shape[0]):
        xb = xbf_ref[i]
        pd1 = jnp.dot(w1_ref[...], xb, preferred_element_type=jnp.float32)
        q1 = _bf16_cat(_vn_layer(pd1, c1, s1_ref[...], ms1_ref[...]))
        pd2 = jnp.dot(w2_ref[...], q1, preferred_element_type=jnp.float32)
        q2 = _bf16_cat(_vn_layer(pd2, c2, s2_ref[...], ms2_ref[...]))
        z = jnp.dot(wl_ref[...], q2,
                    preferred_element_type=jnp.float32)
        z_ref[:, 0, i] = z
        zb = z.astype(jnp.bfloat16)
        for k in range(3):
            xstd_ref[i, k * c:(k + 1) * c] = (
                xb[0:c] * zb[k:k + 1] + xb[c:2 * c] * zb[3 + k:4 + k]
                + xb[2 * c:] * zb[6 + k:7 + k]).astype(jnp.float32)


def _build_weights(wf1, wd1, wf2, wd2, wlin, c, c1, c2):
    i3 = np.eye(3, dtype=np.float32)
    bf = jnp.bfloat16

    def blk(w):
        co, ci = w.shape
        return jnp.einsum('vu,cd->vcud', i3, w).reshape(3 * co, 3 * ci)

    w1f = blk(wf1).astype(bf)
    w1 = jnp.concatenate([blk(wf1), blk(wd1)], axis=0).astype(bf)
    w2f = blk(wf2).astype(bf)
    w2 = jnp.concatenate([blk(wf2), blk(wd2)], axis=0).astype(bf)
    wl = jnp.einsum('vu,kc->vkuc', i3, wlin).reshape(9, 3 * c2).astype(bf)
    return w1f, w1, w2f, w2, wl


def kernel(x, wf1, wd1, wf2, wd2, wlin):
    b, c, v, n = x.shape
    assert v == 3
    c1, c2 = wf1.shape[0], wf2.shape[0]
    f32 = jnp.float32
    xm = jnp.transpose(x.astype(f32), (0, 2, 1, 3)).reshape(b, 3 * c, n)

    w1f, w1, w2f, w2, wl = _build_weights(wf1, wd1, wf2, wd2, wlin, c, c1, c2)

    total = float(b * n)

    bb = 4 if b % 4 == 0 else 1
    g = b // bb
    xspec = pl.BlockSpec((bb, 3 * c, n), lambda i: (i, 0, 0))

    def wspec(a):
        zeros = (0,) * a.ndim
        return pl.BlockSpec(a.shape, lambda i: zeros)

    def momspec(ch):
        return pl.BlockSpec((1, 2 * ch, 1), lambda i: (i, 0, 0))

    params = pltpu.CompilerParams(dimension_semantics=("parallel",),
                                  vmem_limit_bytes=50 * 1024 * 1024)

    def finalize(mom, ch):
        s = jnp.sum(mom[:, :ch, 0], axis=0)
        ss = jnp.sum(mom[:, ch:, 0], axis=0)
        mean = s / total
        var = ss / total - mean * mean
        istd = jax.lax.rsqrt(var + _BN_EPS)
        return istd.reshape(ch, 1), (mean * istd).reshape(ch, 1)

    xbf, mom1 = pl.pallas_call(
        functools.partial(_pass1_kernel, c1=c1),
        grid=(g,),
        in_specs=[xspec, wspec(w1f)],
        out_specs=(xspec, momspec(c1)),
        out_shape=(jax.ShapeDtypeStruct((b, 3 * c, n), jnp.bfloat16),
                   jax.ShapeDtypeStruct((g, 2 * c1, 1), f32)),
        compiler_params=params,
    )(xm, w1f)
    istd1, mistd1 = finalize(mom1, c1)

    mom2 = pl.pallas_call(
        functools.partial(_pass2_kernel, c1=c1, c2=c2),
        grid=(g,),
        in_specs=[xspec, wspec(w1), wspec(istd1), wspec(mistd1), wspec(w2f)],
        out_specs=momspec(c2),
        out_shape=jax.ShapeDtypeStruct((g, 2 * c2, 1), f32),
        compiler_params=params,
    )(xbf, w1, istd1, mistd1, w2f)
    istd2, mistd2 = finalize(mom2, c2)

    xstd_m, z_m = pl.pallas_call(
        functools.partial(_pass3_kernel, c=c, c1=c1, c2=c2),
        grid=(g,),
        in_specs=[xspec, wspec(w1), wspec(istd1), wspec(mistd1), wspec(w2),
                  wspec(istd2), wspec(mistd2), wspec(wl)],
        out_specs=(xspec, pl.BlockSpec((9, 1, bb, n), lambda i: (0, i, 0, 0))),
        out_shape=(jax.ShapeDtypeStruct((b, 3 * c, n), f32),
                   jax.ShapeDtypeStruct((9, g, bb, n), f32)),
        compiler_params=params,
    )(xbf, w1, istd1, mistd1, w2, istd2, mistd2, wl)

    x_std = jnp.transpose(xstd_m.reshape(b, 3, c, n), (0, 2, 1, 3))
    z0 = jnp.transpose(z_m.reshape(3, 3, b, n), (2, 0, 1, 3))
    return x_std, z0
